# baseline (speedup 1.0000x reference)
import functools

import jax
import jax.numpy as jnp
from jax import lax
from jax.experimental import pallas as pl
from jax.experimental.pallas import tpu as pltpu
from jax.experimental.pallas import tpu_sc as plsc

N = 10000
C = 128
K = 27
P = 12000

NPAD = 10240
KC = K * C
KP = K * P
NW = 32
IDX_W = 128
ROWS_PER_W = 80
IDX_ROWS = NW * ROWS_PER_W
NB = 2
GROUPS = ROWS_PER_W // NB
TILE_ROWS = NPAD // 16
DUMMY_ROW = N

_MM_BM = 1024
_MM_BN = 1152


def _mm1_body(x_ref, w_ref, o_ref):
    o_ref[...] = jnp.dot(x_ref[...], w_ref[...],
                         preferred_element_type=jnp.float32)


def _mm1(x, w):
    grid = (NPAD // _MM_BM, KC // _MM_BN)
    return pl.pallas_call(
        _mm1_body,
        grid=grid,
        in_specs=[
            pl.BlockSpec((_MM_BM, C), lambda i, j: (i, 0)),
            pl.BlockSpec((C, _MM_BN), lambda i, j: (0, j)),
        ],
        out_specs=pl.BlockSpec((_MM_BM, _MM_BN), lambda i, j: (i, j)),
        out_shape=jax.ShapeDtypeStruct((NPAD, KC), jnp.float32),
    )(x, w)


def _mm2_body(a0_ref, a1_ref, b_ref, w_ref, o_ref):
    h = jnp.maximum(a0_ref[...] + a1_ref[...] + b_ref[...], 0.0)
    o_ref[...] = jnp.dot(h, w_ref[...], preferred_element_type=jnp.float32)


def _mm2(a0, a1, b, w):
    grid = (NPAD // _MM_BM, KC // _MM_BN)
    return pl.pallas_call(
        _mm2_body,
        grid=grid,
        in_specs=[
            pl.BlockSpec((_MM_BM, C), lambda i, j: (i, 0)),
            pl.BlockSpec((_MM_BM, C), lambda i, j: (i, 0)),
            pl.BlockSpec((1, C), lambda i, j: (0, 0)),
            pl.BlockSpec((C, _MM_BN), lambda i, j: (0, j)),
        ],
        out_specs=pl.BlockSpec((_MM_BM, _MM_BN), lambda i, j: (i, j)),
        out_shape=jax.ShapeDtypeStruct((NPAD, KC), jnp.float32),
    )(a0, a1, b, w)


def _final_body(a0_ref, a1_ref, b_ref, f_ref, o_ref):
    o_ref[...] = jnp.maximum(
        a0_ref[...] + a1_ref[...] + b_ref[...] + f_ref[...], 0.0)


def _final(a0, a1, b, f):
    grid = (NPAD // _MM_BM,)
    blk = pl.BlockSpec((_MM_BM, C), lambda i: (i, 0))
    return pl.pallas_call(
        _final_body,
        grid=grid,
        in_specs=[blk, blk, pl.BlockSpec((1, C), lambda i: (0, 0)), blk],
        out_specs=blk,
        out_shape=jax.ShapeDtypeStruct((NPAD, C), jnp.float32),
    )(a0, a1, b, f)


SLAB = 8
NSLAB = ROWS_PER_W // SLAB


@functools.partial(
    pl.kernel,
    out_type=jax.ShapeDtypeStruct((2, NPAD, C), jnp.float32),
    mesh=plsc.VectorSubcoreMesh(core_axis_name="c", subcore_axis_name="s"),
    scratch_types=[
        pltpu.VMEM((SLAB, 2, IDX_W), jnp.int32),
        pltpu.VMEM((NB, IDX_W, C), jnp.float32),
        pltpu.VMEM_SHARED((NPAD, C), jnp.float32),
        pltpu.SemaphoreType.DMA,
    ],
)
def _sc_gather_scatter(y_hbm, idx_hbm, zeros_hbm, out_hbm,
                       iio_v, rows_v, accum, sem):
    cid = lax.axis_index("c")
    sid = lax.axis_index("s")
    wid = sid * 2 + cid

    pltpu.sync_copy(zeros_hbm.at[pl.ds(sid * TILE_ROWS, TILE_ROWS)],
                    accum.at[pl.ds(sid * TILE_ROWS, TILE_ROWS)])
    plsc.subcore_barrier()

    base = wid * ROWS_PER_W

    def slab(k, carry):
        pltpu.sync_copy(idx_hbm.at[pl.ds(base + k * SLAB, SLAB)], iio_v)

        def group(t, carry2):
            j = t * NB
            cps = [pltpu.async_copy(y_hbm.at[iio_v.at[j + b, 0]],
                                    rows_v.at[b], sem)
                   for b in range(NB)]
            for cp in cps:
                cp.wait()
            for b in range(NB):
                pltpu.sync_copy(rows_v.at[b], accum.at[iio_v.at[j + b, 1]],
                                add=True)
            return carry2

        lax.fori_loop(0, SLAB // NB, group, 0)
        return carry

    lax.fori_loop(0, NSLAB, slab, 0)

    plsc.subcore_barrier()
    pltpu.sync_copy(accum.at[pl.ds(sid * TILE_ROWS, TILE_ROWS)],
                    out_hbm.at[cid, pl.ds(sid * TILE_ROWS, TILE_ROWS)])


def kernel(feats, pairs_in, pairs_out, W1, g1, b1, m1, v1,
           W2, g2, b2, m2, v2):
    eps = 1e-5
    s1 = g1 * lax.rsqrt(v1 + eps)
    s2 = g2 * lax.rsqrt(v2 + eps)
    w1c = (W1 * s1[None, None, :]).transpose(1, 0, 2).reshape(C, KC)
    w2c = (W2 * s2[None, None, :]).transpose(1, 0, 2).reshape(C, KC)
    b1e = (b1 - m1 * s1).reshape(1, C)
    b2e = (b2 - m2 * s2).reshape(1, C)

    featsp = jnp.pad(feats, ((0, NPAD - N), (0, 0)))

    iin = (pairs_in * K + jnp.arange(K, dtype=jnp.int32)[:, None]).reshape(-1)
    iin = jnp.pad(iin, (0, IDX_ROWS * IDX_W - KP)).reshape(IDX_ROWS, IDX_W)
    iout = jnp.pad(pairs_out.reshape(-1), (0, IDX_ROWS * IDX_W - KP),
                   constant_values=DUMMY_ROW).reshape(IDX_ROWS, IDX_W)
    idx = jnp.stack([iin, iout], axis=1)

    zeros = jnp.zeros((NPAD, C), dtype=jnp.float32)

    y1 = _mm1(featsp, w1c).reshape(NPAD * K, C)
    p1 = _sc_gather_scatter(y1, idx, zeros)
    y2 = _mm2(p1[0], p1[1], b1e, w2c).reshape(NPAD * K, C)
    p2 = _sc_gather_scatter(y2, idx, zeros)
    out = _final(p2[0], p2[1], b2e, featsp)
    return out[:N]


# speedup vs baseline: 1.1563x; 1.1563x over previous
import functools

import jax
import jax.numpy as jnp
from jax import lax
from jax.experimental import pallas as pl
from jax.experimental.pallas import tpu as pltpu
from jax.experimental.pallas import tpu_sc as plsc

N = 10000
C = 128
K = 27
P = 12000

NPAD = 10240
KC = K * C
KP = K * P
NW = 32
IDX_W = 128
ROWS_PER_W = 80
IDX_ROWS = NW * ROWS_PER_W
NB = 2
GROUPS = ROWS_PER_W // NB
TILE_ROWS = NPAD // 16
DUMMY_ROW = N

_MM_BM = 1024
_MM_BN = 1152


def _mm1_body(x_ref, w_ref, o_ref):
    o_ref[...] = jnp.dot(x_ref[...], w_ref[...],
                         preferred_element_type=jnp.float32)


def _mm1(x, w):
    grid = (NPAD // _MM_BM, KC // _MM_BN)
    return pl.pallas_call(
        _mm1_body,
        grid=grid,
        in_specs=[
            pl.BlockSpec((_MM_BM, C), lambda i, j: (i, 0)),
            pl.BlockSpec((C, _MM_BN), lambda i, j: (0, j)),
        ],
        out_specs=pl.BlockSpec((_MM_BM, _MM_BN), lambda i, j: (i, j)),
        out_shape=jax.ShapeDtypeStruct((NPAD, KC), jnp.float32),
    )(x, w)


def _mm2_body(a0_ref, a1_ref, b_ref, w_ref, o_ref):
    h = jnp.maximum(a0_ref[...] + a1_ref[...] + b_ref[...], 0.0)
    o_ref[...] = jnp.dot(h, w_ref[...], preferred_element_type=jnp.float32)


def _mm2(a0, a1, b, w):
    grid = (NPAD // _MM_BM, KC // _MM_BN)
    return pl.pallas_call(
        _mm2_body,
        grid=grid,
        in_specs=[
            pl.BlockSpec((_MM_BM, C), lambda i, j: (i, 0)),
            pl.BlockSpec((_MM_BM, C), lambda i, j: (i, 0)),
            pl.BlockSpec((1, C), lambda i, j: (0, 0)),
            pl.BlockSpec((C, _MM_BN), lambda i, j: (0, j)),
        ],
        out_specs=pl.BlockSpec((_MM_BM, _MM_BN), lambda i, j: (i, j)),
        out_shape=jax.ShapeDtypeStruct((NPAD, KC), jnp.float32),
    )(a0, a1, b, w)


def _final_body(a0_ref, a1_ref, b_ref, f_ref, o_ref):
    o_ref[...] = jnp.maximum(
        a0_ref[...] + a1_ref[...] + b_ref[...] + f_ref[...], 0.0)


def _final(a0, a1, b, f):
    grid = (NPAD // _MM_BM,)
    blk = pl.BlockSpec((_MM_BM, C), lambda i: (i, 0))
    return pl.pallas_call(
        _final_body,
        grid=grid,
        in_specs=[blk, blk, pl.BlockSpec((1, C), lambda i: (0, 0)), blk],
        out_specs=blk,
        out_shape=jax.ShapeDtypeStruct((NPAD, C), jnp.float32),
    )(a0, a1, b, f)


CH = 64
STEPS = 2 * ROWS_PER_W
BLOCKS = STEPS // 2


@functools.partial(
    pl.kernel,
    out_type=jax.ShapeDtypeStruct((2, NPAD, C), jnp.float32),
    mesh=plsc.VectorSubcoreMesh(core_axis_name="c", subcore_axis_name="s"),
    scratch_types=[
        pltpu.VMEM((3, 2, 2, CH), jnp.int32),
        pltpu.VMEM((4, CH, C), jnp.float32),
        pltpu.VMEM_SHARED((NPAD, C), jnp.float32),
        pltpu.SemaphoreType.DMA,
        pltpu.SemaphoreType.DMA,
        pltpu.SemaphoreType.DMA,
        pltpu.SemaphoreType.DMA,
        pltpu.SemaphoreType.DMA,
        pltpu.SemaphoreType.DMA,
        pltpu.SemaphoreType.DMA,
        pltpu.SemaphoreType.DMA,
        pltpu.SemaphoreType.DMA,
    ],
)
def _sc_gather_scatter(y_hbm, idx_hbm, zeros_hbm, out_hbm,
                       iio, rows, accum, isem,
                       g0, g1, g2, g3, s0, s1, s2, s3):
    gsems = (g0, g1, g2, g3)
    ssems = (s0, s1, s2, s3)
    cid = lax.axis_index("c")
    sid = lax.axis_index("s")
    wid = sid * 2 + cid

    pltpu.sync_copy(zeros_hbm.at[pl.ds(sid * TILE_ROWS, TILE_ROWS)],
                    accum.at[pl.ds(sid * TILE_ROWS, TILE_ROWS)])
    plsc.subcore_barrier()

    base = wid * STEPS

    def drain(sem_b):
        pltpu.make_async_copy(y_hbm.at[pl.ds(0, CH)], rows.at[0],
                              sem_b).wait()

    def drain_idx():
        pltpu.make_async_copy(idx_hbm.at[pl.ds(0, 2)], iio.at[0],
                              isem).wait()

    def issue_gathers(sl, p):
        for i in range(2):
            b = 2 * p + i
            pltpu.async_copy(y_hbm.at[iio.at[sl, i, 0]], rows.at[b],
                             gsems[b])

    def issue_idx(next_blk, next_sl):
        pltpu.async_copy(idx_hbm.at[pl.ds(base + 2 * next_blk, 2)],
                         iio.at[next_sl], isem)

    def issue_scatters(sl, p):
        for i in range(2):
            b = 2 * p + i
            pltpu.async_copy(rows.at[b], accum.at[iio.at[sl, i, 1]],
                             ssems[b], add=True)

    pltpu.sync_copy(idx_hbm.at[pl.ds(base, 2)], iio.at[0])
    issue_gathers(0, 0)
    issue_idx(1, 1)
    drain_idx()
    issue_gathers(1, 1)
    issue_idx(2, 2)
    drain(g0)
    drain(g1)
    issue_scatters(0, 0)

    def superblock(sb, carry):
        for p in range(2):
            blk = 2 * sb + p
            sl = lax.rem(blk, 3)
            slp = lax.rem(blk - 1, 3)
            q = 1 - p
            drain(ssems[2 * p])
            drain(ssems[2 * p + 1])
            drain_idx()
            issue_gathers(sl, p)
            issue_idx(blk + 1, lax.rem(blk + 1, 3))
            drain(gsems[2 * q])
            drain(gsems[2 * q + 1])
            issue_scatters(slp, q)
        return carry

    lax.fori_loop(1, BLOCKS // 2, superblock, 0)

    drain(g2)
    drain(g3)
    issue_scatters((BLOCKS - 1) % 3, 1)
    for b in range(4):
        drain(ssems[b])
    drain_idx()

    plsc.subcore_barrier()
    pltpu.sync_copy(accum.at[pl.ds(sid * TILE_ROWS, TILE_ROWS)],
                    out_hbm.at[cid, pl.ds(sid * TILE_ROWS, TILE_ROWS)])


def kernel(feats, pairs_in, pairs_out, W1, g1, b1, m1, v1,
           W2, g2, b2, m2, v2):
    eps = 1e-5
    s1 = g1 * lax.rsqrt(v1 + eps)
    s2 = g2 * lax.rsqrt(v2 + eps)
    w1c = (W1 * s1[None, None, :]).transpose(1, 0, 2).reshape(C, KC)
    w2c = (W2 * s2[None, None, :]).transpose(1, 0, 2).reshape(C, KC)
    b1e = (b1 - m1 * s1).reshape(1, C)
    b2e = (b2 - m2 * s2).reshape(1, C)

    featsp = jnp.pad(feats, ((0, NPAD - N), (0, 0)))

    npairs = NW * STEPS * CH
    iin = (pairs_in * K + jnp.arange(K, dtype=jnp.int32)[:, None]).reshape(-1)
    iin = jnp.pad(iin, (0, npairs - KP)).reshape(NW * STEPS, CH)
    iout = jnp.pad(pairs_out.reshape(-1), (0, npairs - KP),
                   constant_values=DUMMY_ROW).reshape(NW * STEPS, CH)
    idx = jnp.pad(jnp.stack([iin, iout], axis=1), ((0, 2), (0, 0), (0, 0)))

    zeros = jnp.zeros((NPAD, C), dtype=jnp.float32)

    y1 = _mm1(featsp, w1c).reshape(NPAD * K, C)
    p1 = _sc_gather_scatter(y1, idx, zeros)
    y2 = _mm2(p1[0], p1[1], b1e, w2c).reshape(NPAD * K, C)
    p2 = _sc_gather_scatter(y2, idx, zeros)
    out = _final(p2[0], p2[1], b2e, featsp)
    return out[:N]


# speedup vs baseline: 1.7495x; 1.5131x over previous
import functools

import jax
import jax.numpy as jnp
from jax import lax
from jax.experimental import pallas as pl
from jax.experimental.pallas import tpu as pltpu
from jax.experimental.pallas import tpu_sc as plsc

N = 10000
C = 128
K = 27
P = 12000

NPAD = 10240
KC = K * C
KP = K * P
NW = 32
IDX_W = 128
ROWS_PER_W = 80
IDX_ROWS = NW * ROWS_PER_W
NB = 2
GROUPS = ROWS_PER_W // NB
TILE_ROWS = NPAD // 16
DUMMY_ROW = N

_MM_BM = 1024
_MM_BN = 1152


def _mm1_body(x_ref, w_ref, o_ref):
    o_ref[...] = jnp.dot(x_ref[...], w_ref[...],
                         preferred_element_type=jnp.float32)


def _mm1(x, w):
    grid = (NPAD // _MM_BM, KC // _MM_BN)
    return pl.pallas_call(
        _mm1_body,
        grid=grid,
        in_specs=[
            pl.BlockSpec((_MM_BM, C), lambda i, j: (i, 0)),
            pl.BlockSpec((C, _MM_BN), lambda i, j: (0, j)),
        ],
        out_specs=pl.BlockSpec((_MM_BM, _MM_BN), lambda i, j: (i, j)),
        out_shape=jax.ShapeDtypeStruct((NPAD, KC), jnp.float32),
    )(x, w)


def _mm2_body(a0_ref, a1_ref, b_ref, w_ref, o_ref):
    h = jnp.maximum(a0_ref[...] + a1_ref[...] + b_ref[...], 0.0)
    o_ref[...] = jnp.dot(h, w_ref[...], preferred_element_type=jnp.float32)


def _mm2(a0, a1, b, w):
    grid = (NPAD // _MM_BM, KC // _MM_BN)
    return pl.pallas_call(
        _mm2_body,
        grid=grid,
        in_specs=[
            pl.BlockSpec((_MM_BM, C), lambda i, j: (i, 0)),
            pl.BlockSpec((_MM_BM, C), lambda i, j: (i, 0)),
            pl.BlockSpec((1, C), lambda i, j: (0, 0)),
            pl.BlockSpec((C, _MM_BN), lambda i, j: (0, j)),
        ],
        out_specs=pl.BlockSpec((_MM_BM, _MM_BN), lambda i, j: (i, j)),
        out_shape=jax.ShapeDtypeStruct((NPAD, KC), jnp.float32),
    )(a0, a1, b, w)


def _final_body(a0_ref, a1_ref, b_ref, f_ref, o_ref):
    o_ref[...] = jnp.maximum(
        a0_ref[...] + a1_ref[...] + b_ref[...] + f_ref[...], 0.0)


def _final(a0, a1, b, f):
    grid = (NPAD // _MM_BM,)
    blk = pl.BlockSpec((_MM_BM, C), lambda i: (i, 0))
    return pl.pallas_call(
        _final_body,
        grid=grid,
        in_specs=[blk, blk, pl.BlockSpec((1, C), lambda i: (0, 0)), blk],
        out_specs=blk,
        out_shape=jax.ShapeDtypeStruct((NPAD, C), jnp.float32),
    )(a0, a1, b, f)


CH = 64
STEPS = 2 * ROWS_PER_W
BLOCKS = STEPS // 2


@functools.partial(
    pl.kernel,
    out_type=jax.ShapeDtypeStruct((2, NPAD, C), jnp.float32),
    mesh=plsc.VectorSubcoreMesh(core_axis_name="c", subcore_axis_name="s"),
    scratch_types=[
        pltpu.VMEM((3, 2, 2, CH), jnp.int32),
        pltpu.VMEM((4, CH, C), jnp.float32),
        pltpu.VMEM_SHARED((NPAD, C), jnp.float32),
        pltpu.SemaphoreType.DMA,
        pltpu.SemaphoreType.DMA,
        pltpu.SemaphoreType.DMA,
        pltpu.SemaphoreType.DMA,
        pltpu.SemaphoreType.DMA,
        pltpu.SemaphoreType.DMA,
        pltpu.SemaphoreType.DMA,
        pltpu.SemaphoreType.DMA,
        pltpu.SemaphoreType.DMA,
    ],
)
def _sc_gather_scatter(y_hbm, idx_hbm, zeros_hbm, out_hbm,
                       iio, rows, accum, isem,
                       g0, g1, g2, g3, s0, s1, s2, s3):
    gsems = (g0, g1, g2, g3)
    ssems = (s0, s1, s2, s3)
    cid = lax.axis_index("c")
    sid = lax.axis_index("s")
    wid = sid * 2 + cid

    pltpu.sync_copy(zeros_hbm.at[pl.ds(sid * TILE_ROWS, TILE_ROWS)],
                    accum.at[pl.ds(sid * TILE_ROWS, TILE_ROWS)])
    plsc.subcore_barrier()

    base = wid * STEPS

    def drain(sem_b):
        pltpu.make_async_copy(y_hbm.at[pl.ds(0, CH)], rows.at[0],
                              sem_b).wait()

    def drain_idx():
        pltpu.make_async_copy(idx_hbm.at[pl.ds(0, 2)], iio.at[0],
                              isem).wait()

    def issue_gathers(sl, p):
        for i in range(2):
            b = 2 * p + i
            pltpu.async_copy(y_hbm.at[iio.at[sl, i, 0]], rows.at[b],
                             gsems[b])

    def issue_idx(next_blk, next_sl):
        pltpu.async_copy(idx_hbm.at[pl.ds(base + 2 * next_blk, 2)],
                         iio.at[next_sl], isem)

    def issue_scatters(sl, p):
        for i in range(2):
            b = 2 * p + i
            pltpu.async_copy(rows.at[b], accum.at[iio.at[sl, i, 1]],
                             ssems[b], add=True)

    pltpu.sync_copy(idx_hbm.at[pl.ds(base, 2)], iio.at[0])
    issue_gathers(0, 0)
    issue_idx(1, 1)
    drain_idx()
    issue_gathers(1, 1)
    issue_idx(2, 2)
    drain(g0)
    drain(g1)
    issue_scatters(0, 0)

    def superblock(sb, carry):
        for p in range(2):
            blk = 2 * sb + p
            sl = lax.rem(blk, 3)
            slp = lax.rem(blk - 1, 3)
            q = 1 - p
            drain(ssems[2 * p])
            drain(ssems[2 * p + 1])
            drain_idx()
            issue_gathers(sl, p)
            issue_idx(blk + 1, lax.rem(blk + 1, 3))
            drain(gsems[2 * q])
            drain(gsems[2 * q + 1])
            issue_scatters(slp, q)
        return carry

    lax.fori_loop(1, BLOCKS // 2, superblock, 0)

    drain(g2)
    drain(g3)
    issue_scatters((BLOCKS - 1) % 3, 1)
    for b in range(4):
        drain(ssems[b])
    drain_idx()

    plsc.subcore_barrier()
    pltpu.sync_copy(accum.at[pl.ds(sid * TILE_ROWS, TILE_ROWS)],
                    out_hbm.at[cid, pl.ds(sid * TILE_ROWS, TILE_ROWS)])


def kernel(feats, pairs_in, pairs_out, W1, g1, b1, m1, v1,
           W2, g2, b2, m2, v2):
    eps = 1e-5
    s1 = g1 * lax.rsqrt(v1 + eps)
    s2 = g2 * lax.rsqrt(v2 + eps)
    w1c = (W1 * s1[None, None, :]).transpose(1, 0, 2).reshape(C, KC)
    w2c = (W2 * s2[None, None, :]).transpose(1, 0, 2).reshape(C, KC)
    b1e = (b1 - m1 * s1).reshape(1, C)
    b2e = (b2 - m2 * s2).reshape(1, C)

    featsp = jnp.pad(feats, ((0, NPAD - N), (0, 0)))

    npairs = NW * STEPS * CH
    npad_pairs = npairs - KP
    pad_ramp = jnp.arange(npad_pairs, dtype=jnp.int32)
    iin = (pairs_in * K + jnp.arange(K, dtype=jnp.int32)[:, None]).reshape(-1)
    iin = jnp.concatenate([iin, (pad_ramp * 4099) % (N * K)])
    iin = iin.reshape(NW * STEPS, CH)
    iout = jnp.concatenate([pairs_out.reshape(-1),
                            DUMMY_ROW + pad_ramp % (NPAD - N)])
    iout = iout.reshape(NW * STEPS, CH)
    idx = jnp.pad(jnp.stack([iin, iout], axis=1), ((0, 2), (0, 0), (0, 0)))

    zeros = jnp.zeros((NPAD, C), dtype=jnp.float32)

    y1 = _mm1(featsp, w1c).reshape(NPAD * K, C)
    p1 = _sc_gather_scatter(y1, idx, zeros)
    y2 = _mm2(p1[0], p1[1], b1e, w2c).reshape(NPAD * K, C)
    p2 = _sc_gather_scatter(y2, idx, zeros)
    out = _final(p2[0], p2[1], b2e, featsp)
    return out[:N]


# speedup vs baseline: 2.8827x; 1.6477x over previous
import functools

import jax
import jax.numpy as jnp
from jax import lax
from jax.experimental import pallas as pl
from jax.experimental.pallas import tpu as pltpu
from jax.experimental.pallas import tpu_sc as plsc

N = 10000
C = 128
K = 27
P = 12000

NPAD = 10240
KC = K * C
KP = K * P
NW = 32
IDX_W = 128
ROWS_PER_W = 80
IDX_ROWS = NW * ROWS_PER_W
NB = 2
GROUPS = ROWS_PER_W // NB
TILE_ROWS = NPAD // 16
DUMMY_ROW = N

_MM_BM = 1024
_MM_BN = 1152


_MM_BK = _MM_BN // C


def _mm1_body(x_ref, w_ref, o_ref):
    for t in range(_MM_BK):
        o_ref[t] = jnp.dot(x_ref[...], w_ref[:, t * C:(t + 1) * C],
                           preferred_element_type=jnp.float32)


def _mm1(x, w):
    grid = (NPAD // _MM_BM, KC // _MM_BN)
    return pl.pallas_call(
        _mm1_body,
        grid=grid,
        in_specs=[
            pl.BlockSpec((_MM_BM, C), lambda i, j: (i, 0)),
            pl.BlockSpec((C, _MM_BN), lambda i, j: (0, j)),
        ],
        out_specs=pl.BlockSpec((_MM_BK, _MM_BM, C), lambda i, j: (j, i, 0)),
        out_shape=jax.ShapeDtypeStruct((K, NPAD, C), jnp.float32),
    )(x, w)


def _mm2_body(a0_ref, a1_ref, b_ref, w_ref, o_ref):
    h = jnp.maximum(a0_ref[...] + a1_ref[...] + b_ref[...], 0.0)
    for t in range(_MM_BK):
        o_ref[t] = jnp.dot(h, w_ref[:, t * C:(t + 1) * C],
                           preferred_element_type=jnp.float32)


def _mm2(a0, a1, b, w):
    grid = (NPAD // _MM_BM, KC // _MM_BN)
    return pl.pallas_call(
        _mm2_body,
        grid=grid,
        in_specs=[
            pl.BlockSpec((_MM_BM, C), lambda i, j: (i, 0)),
            pl.BlockSpec((_MM_BM, C), lambda i, j: (i, 0)),
            pl.BlockSpec((1, C), lambda i, j: (0, 0)),
            pl.BlockSpec((C, _MM_BN), lambda i, j: (0, j)),
        ],
        out_specs=pl.BlockSpec((_MM_BK, _MM_BM, C), lambda i, j: (j, i, 0)),
        out_shape=jax.ShapeDtypeStruct((K, NPAD, C), jnp.float32),
    )(a0, a1, b, w)


def _final_body(a0_ref, a1_ref, b_ref, f_ref, o_ref):
    o_ref[...] = jnp.maximum(
        a0_ref[...] + a1_ref[...] + b_ref[...] + f_ref[...], 0.0)


def _final(a0, a1, b, f):
    grid = (NPAD // _MM_BM,)
    blk = pl.BlockSpec((_MM_BM, C), lambda i: (i, 0))
    return pl.pallas_call(
        _final_body,
        grid=grid,
        in_specs=[blk, blk, pl.BlockSpec((1, C), lambda i: (0, 0)), blk],
        out_specs=blk,
        out_shape=jax.ShapeDtypeStruct((NPAD, C), jnp.float32),
    )(a0, a1, b, f)


CH = 64
STEPS = 2 * ROWS_PER_W
BLOCKS = STEPS // 2


@functools.partial(
    pl.kernel,
    out_type=jax.ShapeDtypeStruct((2, NPAD, C), jnp.float32),
    mesh=plsc.VectorSubcoreMesh(core_axis_name="c", subcore_axis_name="s"),
    scratch_types=[
        pltpu.VMEM((3, 2, 2, CH), jnp.int32),
        pltpu.VMEM((4, CH, C), jnp.float32),
        pltpu.VMEM_SHARED((NPAD, C), jnp.float32),
        pltpu.SemaphoreType.DMA,
        pltpu.SemaphoreType.DMA,
        pltpu.SemaphoreType.DMA,
        pltpu.SemaphoreType.DMA,
        pltpu.SemaphoreType.DMA,
        pltpu.SemaphoreType.DMA,
        pltpu.SemaphoreType.DMA,
        pltpu.SemaphoreType.DMA,
        pltpu.SemaphoreType.DMA,
    ],
)
def _sc_gather_scatter(y_hbm, idx_hbm, zeros_hbm, out_hbm,
                       iio, rows, accum, isem,
                       g0, g1, g2, g3, s0, s1, s2, s3):
    gsems = (g0, g1, g2, g3)
    ssems = (s0, s1, s2, s3)
    cid = lax.axis_index("c")
    sid = lax.axis_index("s")
    wid = sid * 2 + cid

    pltpu.sync_copy(zeros_hbm.at[pl.ds(sid * TILE_ROWS, TILE_ROWS)],
                    accum.at[pl.ds(sid * TILE_ROWS, TILE_ROWS)])
    plsc.subcore_barrier()

    base = wid * STEPS

    def drain(sem_b):
        pltpu.make_async_copy(y_hbm.at[pl.ds(0, CH)], rows.at[0],
                              sem_b).wait()

    def drain_idx():
        pltpu.make_async_copy(idx_hbm.at[pl.ds(0, 2)], iio.at[0],
                              isem).wait()

    def issue_gathers(sl, p):
        for i in range(2):
            b = 2 * p + i
            pltpu.async_copy(y_hbm.at[iio.at[sl, i, 0]], rows.at[b],
                             gsems[b])

    def issue_idx(next_blk, next_sl):
        pltpu.async_copy(idx_hbm.at[pl.ds(base + 2 * next_blk, 2)],
                         iio.at[next_sl], isem)

    def issue_scatters(sl, p):
        for i in range(2):
            b = 2 * p + i
            pltpu.async_copy(rows.at[b], accum.at[iio.at[sl, i, 1]],
                             ssems[b], add=True)

    pltpu.sync_copy(idx_hbm.at[pl.ds(base, 2)], iio.at[0])
    issue_gathers(0, 0)
    issue_idx(1, 1)
    drain_idx()
    issue_gathers(1, 1)
    issue_idx(2, 2)
    drain(g0)
    drain(g1)
    issue_scatters(0, 0)

    def superblock(sb, carry):
        for p in range(2):
            blk = 2 * sb + p
            sl = lax.rem(blk, 3)
            slp = lax.rem(blk - 1, 3)
            q = 1 - p
            drain(ssems[2 * p])
            drain(ssems[2 * p + 1])
            drain_idx()
            issue_gathers(sl, p)
            issue_idx(blk + 1, lax.rem(blk + 1, 3))
            drain(gsems[2 * q])
            drain(gsems[2 * q + 1])
            issue_scatters(slp, q)
        return carry

    lax.fori_loop(1, BLOCKS // 2, superblock, 0)

    drain(g2)
    drain(g3)
    issue_scatters((BLOCKS - 1) % 3, 1)
    for b in range(4):
        drain(ssems[b])
    drain_idx()

    plsc.subcore_barrier()
    pltpu.sync_copy(accum.at[pl.ds(sid * TILE_ROWS, TILE_ROWS)],
                    out_hbm.at[cid, pl.ds(sid * TILE_ROWS, TILE_ROWS)])


def kernel(feats, pairs_in, pairs_out, W1, g1, b1, m1, v1,
           W2, g2, b2, m2, v2):
    eps = 1e-5
    s1 = g1 * lax.rsqrt(v1 + eps)
    s2 = g2 * lax.rsqrt(v2 + eps)
    w1c = (W1 * s1[None, None, :]).transpose(1, 0, 2).reshape(C, KC)
    w2c = (W2 * s2[None, None, :]).transpose(1, 0, 2).reshape(C, KC)
    b1e = (b1 - m1 * s1).reshape(1, C)
    b2e = (b2 - m2 * s2).reshape(1, C)

    featsp = jnp.pad(feats, ((0, NPAD - N), (0, 0)))

    npairs = NW * STEPS * CH
    npad_pairs = npairs - KP
    pad_ramp = jnp.arange(npad_pairs, dtype=jnp.int32)
    iin = (pairs_in
           + (jnp.arange(K, dtype=jnp.int32) * NPAD)[:, None]).reshape(-1)
    iin = jnp.concatenate([iin, (pad_ramp * 4099) % (N * K)])
    iin = iin.reshape(NW * STEPS, CH)
    iout = jnp.concatenate([pairs_out.reshape(-1),
                            DUMMY_ROW + pad_ramp % (NPAD - N)])
    iout = iout.reshape(NW * STEPS, CH)
    idx = jnp.pad(jnp.stack([iin, iout], axis=1), ((0, 2), (0, 0), (0, 0)))

    zeros = jnp.zeros((NPAD, C), dtype=jnp.float32)

    y1 = _mm1(featsp, w1c).reshape(NPAD * K, C)
    p1 = _sc_gather_scatter(y1, idx, zeros)
    y2 = _mm2(p1[0], p1[1], b1e, w2c).reshape(NPAD * K, C)
    p2 = _sc_gather_scatter(y2, idx, zeros)
    out = _final(p2[0], p2[1], b2e, featsp)
    return out[:N]
